# baseline (speedup 1.0000x reference)
import functools

import jax
import jax.numpy as jnp
from jax import lax
from jax.experimental import pallas as pl
from jax.experimental.pallas import tpu as pltpu
from jax.experimental.pallas import tpu_sc as plsc

NC = 2
NS = 16
NW = NC * NS
LANES = 16
NSEQ = 2
NBUF = 2


def _build_sc_call(B, L, V, D):
    seq_per_w = B // NW
    rows_per_w = seq_per_w * L
    crows = NSEQ * L
    nchunks = seq_per_w // NSEQ
    ngroups = nchunks // NBUF
    DW = D // 2

    mesh = plsc.VectorSubcoreMesh(core_axis_name="c", subcore_axis_name="s")

    @functools.partial(
        pl.kernel,
        out_type=jax.ShapeDtypeStruct((B * L, D), jnp.float32),
        mesh=mesh,
        scratch_types=[
            pltpu.VMEM((rows_per_w,), jnp.int32),
            pltpu.VMEM((L, D), jnp.float32),
            [pltpu.VMEM((crows, DW), jnp.int32) for _ in range(NBUF)],
            [pltpu.VMEM((crows, D), jnp.float32) for _ in range(NBUF)],
            [pltpu.SemaphoreType.DMA for _ in range(NBUF)],
            [pltpu.SemaphoreType.DMA for _ in range(NBUF)],
        ],
        compiler_params=pltpu.CompilerParams(use_tc_tiling_on_sc=False, needs_layout_passes=False),
    )
    def sc_fn(x_hbm, pe_hbm, table_hbm, out_hbm,
              idx_v, pe_v, gbufs, obufs, gsems, osems):
        wid = lax.axis_index("s") * NC + lax.axis_index("c")
        row_base = wid * rows_per_w
        pltpu.sync_copy(x_hbm.at[pl.ds(row_base, rows_per_w)], idx_v)
        pltpu.sync_copy(pe_hbm, pe_v)

        def gather(c, b):
            pltpu.async_copy(
                table_hbm.at[idx_v.at[pl.ds(c * crows, crows)]], gbufs[b], gsems[b]
            )

        def wait_gather(b):
            pltpu.make_async_copy(
                table_hbm.at[idx_v.at[pl.ds(0, crows)]], gbufs[b], gsems[b]
            ).wait()

        def scatter(c, b):
            pltpu.async_copy(
                obufs[b], out_hbm.at[pl.ds(row_base + c * crows, crows)], osems[b]
            )

        def wait_scatter(b):
            pltpu.make_async_copy(
                obufs[b], out_hbm.at[pl.ds(row_base, crows)], osems[b]
            ).wait()

        for b in range(NBUF):
            gather(b, b)

        hi_mask = jnp.int32(-65536)

        def grp_body(g, carry):
            for b in range(NBUF):
                c = g * NBUF + b
                wait_gather(b)

                @pl.when(g >= 1)
                def _(b=b):
                    wait_scatter(b)

                def add_body(l, acc, gbuf=gbufs[b], obuf=obufs[b]):
                    pes = [pe_v[l, pl.ds(j * LANES, LANES)] for j in range(4)]
                    for q in range(NSEQ):
                        r = q * L + l
                        for k in range(2):
                            v = gbuf[r, pl.ds(k * LANES, LANES)]
                            fe = plsc.bitcast(v << 16, jnp.float32)
                            fo = plsc.bitcast(v & hi_mask, jnp.float32)
                            obuf[r, pl.ds(2 * k * LANES, LANES)] = fe + pes[2 * k]
                            obuf[r, pl.ds((2 * k + 1) * LANES, LANES)] = (
                                fo + pes[2 * k + 1]
                            )
                    return acc

                lax.fori_loop(0, L, add_body, 0, unroll=2)

                @pl.when(g < ngroups - 1)
                def _(c=c, b=b):
                    gather(c + NBUF, b)

                scatter(c, b)
            return carry

        lax.fori_loop(0, ngroups, grp_body, 0)

        for b in range(NBUF):
            wait_scatter(b)

    return sc_fn


def kernel(x, table, pe):
    B, L = x.shape
    V, D = table.shape
    x_flat = x.reshape(B * L)
    pe_block = pe[0, :L, :]
    t = table.reshape(V, D // 32, 2, 16)
    t = jnp.swapaxes(t, 2, 3).reshape(V, D)
    t16 = t.astype(jnp.bfloat16).reshape(V, D // 2, 2)
    t32 = jax.lax.bitcast_convert_type(t16, jnp.int32)
    sc_fn = _build_sc_call(B, L, V, D)
    out = sc_fn(x_flat, pe_block, t32)
    return out.reshape(B, L, D)


# speedup vs baseline: 1.0006x; 1.0006x over previous
import functools

import jax
import jax.numpy as jnp
from jax import lax
from jax.experimental import pallas as pl
from jax.experimental.pallas import tpu as pltpu
from jax.experimental.pallas import tpu_sc as plsc

NC = 2
NS = 16
NW = NC * NS
LANES = 16
NSEQ = 2
NBUF = 2


def _build_sc_call(B, L, V, D):
    seq_per_w = B // NW
    rows_per_w = seq_per_w * L
    vregs_per_row = D // LANES
    crows = NSEQ * L
    nchunks = seq_per_w // NSEQ
    ngroups = nchunks // NBUF

    mesh = plsc.VectorSubcoreMesh(core_axis_name="c", subcore_axis_name="s")

    @functools.partial(
        pl.kernel,
        out_type=jax.ShapeDtypeStruct((B * L, D), jnp.float32),
        mesh=mesh,
        scratch_types=[
            pltpu.VMEM((rows_per_w,), jnp.int32),
            pltpu.VMEM((L, D), jnp.float32),
            [pltpu.VMEM((crows, D), jnp.float32) for _ in range(NBUF)],
            [pltpu.SemaphoreType.DMA for _ in range(NBUF)],
            [pltpu.SemaphoreType.DMA for _ in range(NBUF)],
        ],
        compiler_params=pltpu.CompilerParams(use_tc_tiling_on_sc=False),
    )
    def sc_fn(x_hbm, pe_hbm, table_hbm, out_hbm, idx_v, pe_v, bufs, gsems, osems):
        wid = lax.axis_index("s") * NC + lax.axis_index("c")
        row_base = wid * rows_per_w
        pltpu.sync_copy(x_hbm.at[pl.ds(row_base, rows_per_w)], idx_v)
        pltpu.sync_copy(pe_hbm, pe_v)

        def gather(c, b):
            pltpu.async_copy(
                table_hbm.at[idx_v.at[pl.ds(c * crows, crows)]], bufs[b], gsems[b]
            )

        def wait_gather(b):
            pltpu.make_async_copy(
                table_hbm.at[idx_v.at[pl.ds(0, crows)]], bufs[b], gsems[b]
            ).wait()

        def scatter(c, b):
            pltpu.async_copy(
                bufs[b], out_hbm.at[pl.ds(row_base + c * crows, crows)], osems[b]
            )

        def wait_scatter(b):
            pltpu.make_async_copy(
                bufs[b], out_hbm.at[pl.ds(row_base, crows)], osems[b]
            ).wait()

        for b in range(NBUF):
            gather(b, b)

        def grp_body(g, carry):
            for b in range(NBUF):
                c = g * NBUF + b
                wait_gather(b)

                def add_body(l, acc, buf=bufs[b]):
                    for j in range(vregs_per_row):
                        sl = pl.ds(j * LANES, LANES)
                        pej = pe_v[l, sl]
                        for q in range(NSEQ):
                            r = q * L + l
                            buf[r, sl] = buf[r, sl] + pej
                    return acc

                lax.fori_loop(0, L, add_body, 0, unroll=2)

                scatter(c, b)

                @pl.when(g < ngroups - 1)
                def _(c=c, b=b):
                    wait_scatter(b)
                    gather(c + NBUF, b)
            return carry

        lax.fori_loop(0, ngroups, grp_body, 0)

        for b in range(NBUF):
            wait_scatter(b)

    return sc_fn


def kernel(x, table, pe):
    B, L = x.shape
    V, D = table.shape
    x_flat = x.reshape(B * L)
    pe_block = pe[0, :L, :]
    sc_fn = _build_sc_call(B, L, V, D)
    out = sc_fn(x_flat, pe_block, table)
    return out.reshape(B, L, D)
